# Initial kernel scaffold; baseline (speedup 1.0000x reference)
import jax
import jax.numpy as jnp
from jax.experimental import pallas as pl


def kernel(x, user_table, genre_table, gamma, beta):
    raise NotImplementedError("write your pallas kernel here")


# speedup vs baseline: 9.1647x; 9.1647x over previous
import jax
import jax.numpy as jnp
from jax import lax
from jax.experimental import pallas as pl
from jax.experimental.pallas import tpu as pltpu
from jax.experimental.pallas import tpu_sc as plsc

NC = 2
NS = 16
LANES = 16
NW = NC * NS

B = 4096
H = 50
D = 64
BPW = B // NW
DCH = D // LANES
EPS = 1e-5


def _pool_body(xt_hbm, user_hbm, genre_hbm, out_hbm,
               idx_v, buf0, buf1, acc, sem0, sem1):
    wid = lax.axis_index("s") * NC + lax.axis_index("c")
    base = wid * BPW

    for t, table in ((0, user_hbm), (1, genre_hbm)):
        pltpu.sync_copy(xt_hbm.at[t, :, pl.ds(base, BPW)], idx_v)

        def _zero(i, carry):
            for j in range(DCH):
                acc[i, pl.ds(j * LANES, LANES)] = jnp.zeros((LANES,), jnp.float32)
            return carry
        lax.fori_loop(0, BPW, _zero, None)

        pltpu.async_copy(table.at[idx_v.at[0]], buf0, sem0)
        pltpu.async_copy(table.at[idx_v.at[1]], buf1, sem1)

        def _accum(buf):
            def body(i, carry):
                for j in range(DCH):
                    v = buf[i, pl.ds(j * LANES, LANES)]
                    plsc.addupdate(acc.at[i, pl.ds(j * LANES, LANES)], v)
                return carry
            lax.fori_loop(0, BPW, body, None, unroll=2)

        def _pair(p, carry):
            h0 = 2 * p
            pltpu.make_async_copy(table.at[idx_v.at[h0]], buf0, sem0).wait()
            _accum(buf0)

            @pl.when(h0 + 2 < H)
            def _():
                pltpu.async_copy(table.at[idx_v.at[h0 + 2]], buf0, sem0)

            pltpu.make_async_copy(table.at[idx_v.at[h0 + 1]], buf1, sem1).wait()
            _accum(buf1)

            @pl.when(h0 + 3 < H)
            def _():
                pltpu.async_copy(table.at[idx_v.at[h0 + 3]], buf1, sem1)
            return carry
        lax.fori_loop(0, H // 2, _pair, None)

        pltpu.sync_copy(acc, out_hbm.at[t, pl.ds(base, BPW)])


def _pool(xt, user_table, genre_table):
    mesh = plsc.VectorSubcoreMesh(core_axis_name="c", subcore_axis_name="s",
                                  num_cores=NC, num_subcores=NS)
    return pl.kernel(
        _pool_body,
        out_type=jax.ShapeDtypeStruct((2, B, D), jnp.float32),
        mesh=mesh,
        scratch_types=[
            pltpu.VMEM((H, BPW), jnp.int32),
            pltpu.VMEM((BPW, D), jnp.float32),
            pltpu.VMEM((BPW, D), jnp.float32),
            pltpu.VMEM((BPW, D), jnp.float32),
            pltpu.SemaphoreType.DMA,
            pltpu.SemaphoreType.DMA,
        ],
        compiler_params=pltpu.CompilerParams(use_tc_tiling_on_sc=False),
    )(xt, user_table, genre_table)


def _bn_dot_body(emb_ref, gamma_ref, beta_ref, out_ref):
    gamma = gamma_ref[...]
    beta = beta_ref[...]

    def bn(h):
        m = jnp.mean(h, axis=0, keepdims=True)
        v = jnp.mean((h - m) ** 2, axis=0, keepdims=True)
        return (h - m) * lax.rsqrt(v + EPS) * gamma + beta

    u = bn(emb_ref[0] * (1.0 / H))
    g = bn(emb_ref[1] * (1.0 / H))
    out_ref[...] = jax.nn.sigmoid(jnp.sum(u * g, axis=1))


def _bn_dot(pooled, gamma, beta):
    return pl.pallas_call(
        _bn_dot_body,
        out_shape=jax.ShapeDtypeStruct((B,), jnp.float32),
    )(pooled, gamma, beta)


def kernel(x, user_table, genre_table, gamma, beta):
    xt = jnp.transpose(x.astype(jnp.int32), (0, 2, 1))
    pooled = _pool(xt, user_table, genre_table)
    return _bn_dot(pooled, gamma.reshape(1, D), beta.reshape(1, D))


# speedup vs baseline: 9.3526x; 1.0205x over previous
import jax
import jax.numpy as jnp
from jax import lax
from jax.experimental import pallas as pl
from jax.experimental.pallas import tpu as pltpu
from jax.experimental.pallas import tpu_sc as plsc

NC = 2
NS = 16
LANES = 16
NW = NC * NS

B = 4096
H = 50
D = 64
BPW = B // NW
ROWS2 = BPW // 2
EPS = 1e-5


def _pool_body(xt_hbm, user_hbm, genre_hbm, out_hbm,
               idx_v, buf0, buf1, acc, sem0, sem1):
    wid = lax.axis_index("s") * NC + lax.axis_index("c")

    for t, table in ((0, user_hbm), (1, genre_hbm)):
        pltpu.sync_copy(xt_hbm.at[t, :, wid], idx_v)

        def _zero(p, carry):
            for j in range(8):
                acc[p, pl.ds(j * LANES, LANES)] = jnp.zeros((LANES,), jnp.float32)
            return carry
        lax.fori_loop(0, ROWS2, _zero, None)

        pltpu.async_copy(table.at[idx_v.at[0]], buf0, sem0)
        pltpu.async_copy(table.at[idx_v.at[1]], buf1, sem1)

        def _accum(buf):
            def body(p, carry):
                for jj in range(8):
                    v = buf[2 * p + jj // 4, pl.ds((jj % 4) * LANES, LANES)]
                    plsc.addupdate(acc.at[p, pl.ds(jj * LANES, LANES)], v)
                return carry
            lax.fori_loop(0, ROWS2, body, None, unroll=2)

        def _pair(pr, carry):
            h0 = 2 * pr
            pltpu.make_async_copy(table.at[idx_v.at[h0]], buf0, sem0).wait()
            _accum(buf0)

            @pl.when(h0 + 2 < H)
            def _():
                pltpu.async_copy(table.at[idx_v.at[h0 + 2]], buf0, sem0)

            pltpu.make_async_copy(table.at[idx_v.at[h0 + 1]], buf1, sem1).wait()
            _accum(buf1)

            @pl.when(h0 + 3 < H)
            def _():
                pltpu.async_copy(table.at[idx_v.at[h0 + 3]], buf1, sem1)
            return carry
        lax.fori_loop(0, H // 2, _pair, None)

        pltpu.sync_copy(acc, out_hbm.at[t, pl.ds(wid * ROWS2, ROWS2)])


def _pool(xt, user_table, genre_table):
    mesh = plsc.VectorSubcoreMesh(core_axis_name="c", subcore_axis_name="s",
                                  num_cores=NC, num_subcores=NS)
    return pl.kernel(
        _pool_body,
        out_type=jax.ShapeDtypeStruct((2, B // 2, 128), jnp.float32),
        mesh=mesh,
        scratch_types=[
            pltpu.VMEM((H, BPW), jnp.int32),
            pltpu.VMEM((BPW, D), jnp.float32),
            pltpu.VMEM((BPW, D), jnp.float32),
            pltpu.VMEM((ROWS2, 128), jnp.float32),
            pltpu.SemaphoreType.DMA,
            pltpu.SemaphoreType.DMA,
        ],
        compiler_params=pltpu.CompilerParams(use_tc_tiling_on_sc=False),
    )(xt, user_table, genre_table)


def _bn_dot_body(emb_ref, gamma_ref, beta_ref, out_ref):
    gamma = gamma_ref[...]
    beta = beta_ref[...]

    def bn(h):
        n = 2.0 / B
        m = jnp.sum(h, axis=0, keepdims=True) * n
        sq = jnp.sum(h * h, axis=0, keepdims=True) * n
        mu = (m[:, :D] + m[:, D:]) * 0.5
        var = (sq[:, :D] + sq[:, D:]) * 0.5 - mu * mu
        a = gamma * lax.rsqrt(var + EPS)
        b = beta - a * mu
        a2 = jnp.concatenate([a, a], axis=1)
        b2 = jnp.concatenate([b, b], axis=1)
        return h * a2 + b2

    u = bn(emb_ref[0] * (1.0 / H))
    g = bn(emb_ref[1] * (1.0 / H))
    prod = u * g
    z0 = jnp.sum(prod[:, :D], axis=1, keepdims=True)
    z1 = jnp.sum(prod[:, D:], axis=1, keepdims=True)
    out_ref[...] = jax.nn.sigmoid(jnp.concatenate([z0, z1], axis=1))


def _bn_dot(pooled, gamma, beta):
    return pl.pallas_call(
        _bn_dot_body,
        out_shape=jax.ShapeDtypeStruct((B // 2, 2), jnp.float32),
    )(pooled, gamma, beta)


def kernel(x, user_table, genre_table, gamma, beta):
    xt = jnp.transpose(x.astype(jnp.int32), (0, 2, 1)).reshape(2, H, NW, 128)
    pooled = _pool(xt, user_table, genre_table)
    z = _bn_dot(pooled, gamma.reshape(1, D), beta.reshape(1, D))
    return z.reshape(B)
